# Initial kernel scaffold; baseline (speedup 1.0000x reference)
import jax
import jax.numpy as jnp
from jax.experimental import pallas as pl


def kernel(x, W0, b0, gamma0, beta0, W1, b1, gamma1, beta1, W2, b2, gamma2, beta2, W3, b3, gamma3, beta3):
    raise NotImplementedError("write your pallas kernel here")


# speedup vs baseline: 7.0820x; 7.0820x over previous
import jax
import jax.numpy as jnp
from jax.experimental import pallas as pl
from jax.experimental.pallas import tpu as pltpu

_BN_EPS = 1e-5

_INV_PI = 0.3183098861837907
_PI_HI = 3.140625
_PI_LO = 9.67653589793331e-4
_S3 = -1.0 / 6.0
_S5 = 1.0 / 120.0
_S7 = -1.0 / 5040.0
_S9 = 1.0 / 362880.0


def _fast_sin(v):
    k = jnp.round(v * _INV_PI)
    r = v - k * _PI_HI
    r = r - k * _PI_LO
    r2 = r * r
    p = _S9
    p = p * r2 + _S7
    p = p * r2 + _S5
    p = p * r2 + _S3
    s = r + r * (r2 * p)
    sbits = jax.lax.shift_left(k.astype(jnp.int32), 31)
    return jax.lax.bitcast_convert_type(
        jax.lax.bitcast_convert_type(s, jnp.int32) ^ sbits, jnp.float32)


def _stats(y, s_ref, q_ref):
    s_ref[...] = jnp.sum(y, axis=1, keepdims=True)
    q_ref[...] = jnp.sum(y * y, axis=1, keepdims=True)


def _k_first(x_ref, w_ref, s_ref, q_ref):
    y = jnp.dot(w_ref[...], x_ref[...], preferred_element_type=jnp.float32)
    _stats(y, s_ref, q_ref)


def _k_mid(prev_ref, wf_ref, sh_ref, wn_ref, act_ref, s_ref, q_ref):
    a = _fast_sin(jnp.dot(wf_ref[...], prev_ref[...],
                          preferred_element_type=jnp.float32) + sh_ref[...])
    act_ref[...] = a
    y = jnp.dot(wn_ref[...], a, preferred_element_type=jnp.float32)
    _stats(y, s_ref, q_ref)


def _k_last(prev_ref, wf_ref, sh_ref, o_ref):
    o_ref[...] = _fast_sin(
        jnp.dot(wf_ref[...], prev_ref[...],
                preferred_element_type=jnp.float32) + sh_ref[...]
    ).astype(o_ref.dtype)


def _finalize(s, q, gamma, beta, m_total):
    inv_m = 1.0 / float(m_total)
    mean = jnp.sum(s, axis=0) * inv_m
    ey2 = jnp.sum(q, axis=0) * inv_m
    var = jnp.maximum(ey2 - mean * mean, 0.0)
    scale = gamma * jax.lax.rsqrt(var + _BN_EPS)
    shift = beta - mean * scale
    return scale, shift


def _cparams():
    return pltpu.CompilerParams(dimension_semantics=("parallel",),
                                vmem_limit_bytes=64 * 1024 * 1024)


def _sq3(c, n):
    return pl.BlockSpec((pl.Squeezed(), c, n), lambda b: (b, 0, 0))


def _full(arr):
    return pl.BlockSpec(tuple(int(s) for s in arr.shape), lambda b: (0, 0))


def kernel(x, W0, b0, gamma0, beta0, W1, b1, gamma1, beta1,
           W2, b2, gamma2, beta2, W3, b3, gamma3, beta3):
    B, C0, N = (int(s) for s in x.shape)
    weights = [W0, W1, W2, W3]
    gammas = [gamma0, gamma1, gamma2, gamma3]
    betas = [beta0, beta1, beta2, beta3]
    couts = [int(w.shape[0]) for w in weights]
    cins = [int(w.shape[1]) for w in weights]
    assert cins[0] == C0 and N % 128 == 0 and C0 % 8 == 0
    m_total = B * N
    n_layers = len(weights)

    def stat_outs(c):
        shapes = [jax.ShapeDtypeStruct((B, c, 1), jnp.float32)] * 2
        specs = [_sq3(c, 1)] * 2
        return shapes, specs

    st_sh, st_sp = stat_outs(couts[0])
    s, q = pl.pallas_call(
        _k_first,
        grid=(B,),
        in_specs=[_sq3(C0, N), _full(W0)],
        out_shape=st_sh,
        out_specs=st_sp,
        compiler_params=_cparams(),
        cost_estimate=pl.CostEstimate(
            flops=2 * B * N * C0 * couts[0] + 3 * B * N * couts[0],
            transcendentals=0,
            bytes_accessed=4 * B * C0 * N),
    )(x, W0)

    prev = x
    for i in range(n_layers - 1):
        scale, shift = _finalize(s, q, gammas[i], betas[i], m_total)
        w_eff = weights[i] * scale
        ci, co, cn = cins[i], couts[i], couts[i + 1]
        st_sh, st_sp = stat_outs(cn)
        prev, s, q = pl.pallas_call(
            _k_mid,
            grid=(B,),
            in_specs=[_sq3(ci, N), _full(w_eff), _full(shift),
                      _full(weights[i + 1])],
            out_shape=[jax.ShapeDtypeStruct((B, co, N), jnp.float32)] + st_sh,
            out_specs=[_sq3(co, N)] + st_sp,
            compiler_params=_cparams(),
            cost_estimate=pl.CostEstimate(
                flops=2 * B * N * (ci * co + co * cn) + 11 * B * N * co,
                transcendentals=B * N * co,
                bytes_accessed=4 * B * N * (ci + co)),
        )(prev, w_eff, shift, weights[i + 1])

    scale, shift = _finalize(s, q, gammas[-1], betas[-1], m_total)
    w_eff = weights[-1] * scale
    ci, co = cins[-1], couts[-1]
    out = pl.pallas_call(
        _k_last,
        grid=(B,),
        in_specs=[_sq3(ci, N), _full(w_eff), _full(shift)],
        out_specs=_sq3(co, N),
        out_shape=jax.ShapeDtypeStruct((B, co, N), x.dtype),
        compiler_params=_cparams(),
        cost_estimate=pl.CostEstimate(
            flops=2 * B * N * ci * co + 11 * B * N * co,
            transcendentals=B * N * co,
            bytes_accessed=4 * B * N * (ci + co)),
    )(prev, w_eff, shift)
    return out


# speedup vs baseline: 8.1336x; 1.1485x over previous
import jax
import jax.numpy as jnp
from jax.experimental import pallas as pl
from jax.experimental.pallas import tpu as pltpu

_BN_EPS = 1e-5

_INV_PI = 0.3183098861837907
_PI_F32 = 3.14159265
_S3 = -0.16666667
_S5 = 8.3321608e-3
_S7 = -1.9515296e-4


def _fast_sin(v):
    k = jnp.round(v * _INV_PI)
    r = v - k * _PI_F32
    r2 = r * r
    p = _S7
    p = p * r2 + _S5
    p = p * r2 + _S3
    s = r + r * (r2 * p)
    sbits = jax.lax.shift_left(k.astype(jnp.int32), 31)
    return jax.lax.bitcast_convert_type(
        jax.lax.bitcast_convert_type(s, jnp.int32) ^ sbits, jnp.float32)


def _acc_stats(y, s_ref, q_ref):
    b = pl.program_id(1)
    sums = jnp.sum(y, axis=1, keepdims=True)
    sq = jnp.sum(y * y, axis=1, keepdims=True)

    @pl.when(b == 0)
    def _init():
        s_ref[...] = sums
        q_ref[...] = sq

    @pl.when(b > 0)
    def _acc():
        s_ref[...] += sums
        q_ref[...] += sq


def _bn(sp_ref, qp_ref, g_ref, b_ref, inv_m):
    mean = jnp.sum(sp_ref[...], axis=0) * inv_m
    ey2 = jnp.sum(qp_ref[...], axis=0) * inv_m
    var = jnp.maximum(ey2 - mean * mean, 0.0)
    scale = g_ref[...] * jax.lax.rsqrt(var + _BN_EPS)
    shift = b_ref[...] - mean * scale
    return scale, shift


def _k_first(x_ref, w_ref, s_ref, q_ref):
    y = jnp.dot(w_ref[...], x_ref[...], preferred_element_type=jnp.float32)
    _acc_stats(y, s_ref, q_ref)


def _make_mid(inv_m):
    def _k_mid(prev_ref, wf_ref, g_ref, b_ref, sp_ref, qp_ref, wn_ref,
               act_ref, s_ref, q_ref):
        scale, shift = _bn(sp_ref, qp_ref, g_ref, b_ref, inv_m)
        w_eff = wf_ref[...] * scale
        a = _fast_sin(jnp.dot(w_eff, prev_ref[...].astype(jnp.float32),
                              preferred_element_type=jnp.float32) + shift)
        act_ref[...] = a.astype(act_ref.dtype)
        y = jnp.dot(wn_ref[...], a, preferred_element_type=jnp.float32)
        _acc_stats(y, s_ref, q_ref)
    return _k_mid


def _make_last(inv_m):
    def _k_last(prev_ref, wf_ref, g_ref, b_ref, sp_ref, qp_ref, o_ref):
        scale, shift = _bn(sp_ref, qp_ref, g_ref, b_ref, inv_m)
        w_eff = wf_ref[...] * scale
        o_ref[...] = _fast_sin(
            jnp.dot(w_eff, prev_ref[...].astype(jnp.float32),
                    preferred_element_type=jnp.float32) + shift
        ).astype(o_ref.dtype)
    return _k_last


def _cparams():
    return pltpu.CompilerParams(dimension_semantics=("parallel", "arbitrary"),
                                vmem_limit_bytes=64 * 1024 * 1024)


def kernel(x, W0, b0, gamma0, beta0, W1, b1, gamma1, beta1,
           W2, b2, gamma2, beta2, W3, b3, gamma3, beta3):
    B, C0, N = (int(s) for s in x.shape)
    weights = [W0, W1, W2, W3]
    gammas = [gamma0, gamma1, gamma2, gamma3]
    betas = [beta0, beta1, beta2, beta3]
    couts = [int(w.shape[0]) for w in weights]
    cins = [int(w.shape[1]) for w in weights]
    assert cins[0] == C0 and N % 128 == 0 and C0 % 8 == 0 and B % 2 == 0
    inv_m = 1.0 / float(B * N)
    n_layers = len(weights)
    half = B // 2
    grid = (2, half)

    def _sq3(c, n):
        return pl.BlockSpec((pl.Squeezed(), c, n),
                            lambda cc, b: (cc * half + b, 0, 0))

    def _full(arr):
        shape = tuple(int(s) for s in arr.shape)
        return pl.BlockSpec(shape, lambda cc, b: (0,) * len(shape))

    def _core_col(c):
        return pl.BlockSpec((pl.Squeezed(), c, 1), lambda cc, b: (cc, 0, 0))

    def stat_outs(c):
        shapes = [jax.ShapeDtypeStruct((2, c, 1), jnp.float32)] * 2
        specs = [_core_col(c)] * 2
        return shapes, specs

    st_sh, st_sp = stat_outs(couts[0])
    s, q = pl.pallas_call(
        _k_first,
        grid=grid,
        in_specs=[_sq3(C0, N), _full(W0)],
        out_shape=st_sh,
        out_specs=st_sp,
        compiler_params=_cparams(),
        cost_estimate=pl.CostEstimate(
            flops=2 * B * N * C0 * couts[0] + 3 * B * N * couts[0],
            transcendentals=0,
            bytes_accessed=4 * B * C0 * N),
    )(x, W0)

    prev = x
    for i in range(n_layers - 1):
        ci, co, cn = cins[i], couts[i], couts[i + 1]
        st_sh, st_sp = stat_outs(cn)
        prev, s, q = pl.pallas_call(
            _make_mid(inv_m),
            grid=grid,
            in_specs=[_sq3(ci, N), _full(weights[i]), _full(gammas[i]),
                      _full(betas[i]), _full(s), _full(q),
                      _full(weights[i + 1])],
            out_shape=[jax.ShapeDtypeStruct((B, co, N), jnp.bfloat16)] + st_sh,
            out_specs=[_sq3(co, N)] + st_sp,
            compiler_params=_cparams(),
            cost_estimate=pl.CostEstimate(
                flops=2 * B * N * (ci * co + co * cn) + 11 * B * N * co,
                transcendentals=B * N * co,
                bytes_accessed=4 * B * N * ci // (1 if i == 0 else 2)
                + 2 * B * N * co),
        )(prev, weights[i], gammas[i], betas[i], s, q, weights[i + 1])

    ci, co = cins[-1], couts[-1]
    out = pl.pallas_call(
        _make_last(inv_m),
        grid=grid,
        in_specs=[_sq3(ci, N), _full(weights[-1]), _full(gammas[-1]),
                  _full(betas[-1]), _full(s), _full(q)],
        out_specs=_sq3(co, N),
        out_shape=jax.ShapeDtypeStruct((B, co, N), x.dtype),
        compiler_params=_cparams(),
        cost_estimate=pl.CostEstimate(
            flops=2 * B * N * ci * co + 11 * B * N * co,
            transcendentals=B * N * co,
            bytes_accessed=2 * B * N * ci + 4 * B * N * co),
    )(prev, weights[-1], gammas[-1], betas[-1], s, q)
    return out


# speedup vs baseline: 9.6156x; 1.1822x over previous
import jax
import jax.numpy as jnp
from jax.experimental import pallas as pl
from jax.experimental.pallas import tpu as pltpu

_BN_EPS = 1e-5

_INV_PI = 0.3183098861837907
_PI_F32 = 3.14159265
_S3 = -0.16666667
_S5 = 8.3321608e-3
_S7 = -1.9515296e-4


def _fast_sin(v):
    k = jnp.round(v * _INV_PI)
    r = v - k * _PI_F32
    r2 = r * r
    p = _S7
    p = p * r2 + _S5
    p = p * r2 + _S3
    s = r + r * (r2 * p)
    sbits = jax.lax.shift_left(k.astype(jnp.int32), 31)
    return jax.lax.bitcast_convert_type(
        jax.lax.bitcast_convert_type(s, jnp.int32) ^ sbits, jnp.float32)


_BB = 4


def _acc_stats(sums, sq, s_ref, q_ref):
    b = pl.program_id(1)

    @pl.when(b == 0)
    def _init():
        s_ref[...] = sums
        q_ref[...] = sq

    @pl.when(b > 0)
    def _acc():
        s_ref[...] += sums
        q_ref[...] += sq


def _bn(sp_ref, qp_ref, g_ref, b_ref, inv_m):
    mean = jnp.sum(sp_ref[...], axis=0) * inv_m
    ey2 = jnp.sum(qp_ref[...], axis=0) * inv_m
    var = jnp.maximum(ey2 - mean * mean, 0.0)
    scale = g_ref[...] * jax.lax.rsqrt(var + _BN_EPS)
    shift = b_ref[...] - mean * scale
    return scale, shift


def _k_first(x_ref, w_ref, s_ref, q_ref):
    w = w_ref[...]
    sums, sq = 0.0, 0.0
    for j in range(_BB):
        y = jnp.dot(w, x_ref[j], preferred_element_type=jnp.float32)
        sums += jnp.sum(y, axis=1, keepdims=True)
        sq += jnp.sum(y * y, axis=1, keepdims=True)
    _acc_stats(sums, sq, s_ref, q_ref)


def _make_mid(inv_m):
    def _k_mid(prev_ref, wf_ref, g_ref, b_ref, sp_ref, qp_ref, wn_ref,
               act_ref, s_ref, q_ref):
        scale, shift = _bn(sp_ref, qp_ref, g_ref, b_ref, inv_m)
        w_eff = wf_ref[...] * scale
        wn = wn_ref[...]
        sums, sq = 0.0, 0.0
        for j in range(_BB):
            a = _fast_sin(
                jnp.dot(w_eff, prev_ref[j].astype(jnp.float32),
                        preferred_element_type=jnp.float32) + shift)
            act_ref[j] = a.astype(act_ref.dtype)
            y = jnp.dot(wn, a, preferred_element_type=jnp.float32)
            sums += jnp.sum(y, axis=1, keepdims=True)
            sq += jnp.sum(y * y, axis=1, keepdims=True)
        _acc_stats(sums, sq, s_ref, q_ref)
    return _k_mid


def _make_last(inv_m):
    def _k_last(prev_ref, wf_ref, g_ref, b_ref, sp_ref, qp_ref, o_ref):
        scale, shift = _bn(sp_ref, qp_ref, g_ref, b_ref, inv_m)
        w_eff = wf_ref[...] * scale
        for j in range(_BB):
            o_ref[j] = _fast_sin(
                jnp.dot(w_eff, prev_ref[j].astype(jnp.float32),
                        preferred_element_type=jnp.float32) + shift
            ).astype(o_ref.dtype)
    return _k_last


def _cparams():
    return pltpu.CompilerParams(dimension_semantics=("parallel", "arbitrary"),
                                vmem_limit_bytes=64 * 1024 * 1024)


def kernel(x, W0, b0, gamma0, beta0, W1, b1, gamma1, beta1,
           W2, b2, gamma2, beta2, W3, b3, gamma3, beta3):
    B, C0, N = (int(s) for s in x.shape)
    weights = [W0, W1, W2, W3]
    gammas = [gamma0, gamma1, gamma2, gamma3]
    betas = [beta0, beta1, beta2, beta3]
    couts = [int(w.shape[0]) for w in weights]
    cins = [int(w.shape[1]) for w in weights]
    assert cins[0] == C0 and N % 128 == 0 and C0 % 8 == 0 and B % (2 * _BB) == 0
    inv_m = 1.0 / float(B * N)
    n_layers = len(weights)
    half = B // (2 * _BB)
    grid = (2, half)

    def _sq3(c, n):
        return pl.BlockSpec((_BB, c, n),
                            lambda cc, b: (cc * half + b, 0, 0))

    def _full(arr):
        shape = tuple(int(s) for s in arr.shape)
        return pl.BlockSpec(shape, lambda cc, b: (0,) * len(shape))

    def _core_col(c):
        return pl.BlockSpec((pl.Squeezed(), c, 1), lambda cc, b: (cc, 0, 0))

    def stat_outs(c):
        shapes = [jax.ShapeDtypeStruct((2, c, 1), jnp.float32)] * 2
        specs = [_core_col(c)] * 2
        return shapes, specs

    st_sh, st_sp = stat_outs(couts[0])
    s, q = pl.pallas_call(
        _k_first,
        grid=grid,
        in_specs=[_sq3(C0, N), _full(W0)],
        out_shape=st_sh,
        out_specs=st_sp,
        compiler_params=_cparams(),
        cost_estimate=pl.CostEstimate(
            flops=2 * B * N * C0 * couts[0] + 3 * B * N * couts[0],
            transcendentals=0,
            bytes_accessed=4 * B * C0 * N),
    )(x, W0)

    prev = x
    for i in range(n_layers - 1):
        ci, co, cn = cins[i], couts[i], couts[i + 1]
        st_sh, st_sp = stat_outs(cn)
        prev, s, q = pl.pallas_call(
            _make_mid(inv_m),
            grid=grid,
            in_specs=[_sq3(ci, N), _full(weights[i]), _full(gammas[i]),
                      _full(betas[i]), _full(s), _full(q),
                      _full(weights[i + 1])],
            out_shape=[jax.ShapeDtypeStruct((B, co, N), jnp.bfloat16)] + st_sh,
            out_specs=[_sq3(co, N)] + st_sp,
            compiler_params=_cparams(),
            cost_estimate=pl.CostEstimate(
                flops=2 * B * N * (ci * co + co * cn) + 11 * B * N * co,
                transcendentals=B * N * co,
                bytes_accessed=4 * B * N * ci // (1 if i == 0 else 2)
                + 2 * B * N * co),
        )(prev, weights[i], gammas[i], betas[i], s, q, weights[i + 1])

    ci, co = cins[-1], couts[-1]
    out = pl.pallas_call(
        _make_last(inv_m),
        grid=grid,
        in_specs=[_sq3(ci, N), _full(weights[-1]), _full(gammas[-1]),
                  _full(betas[-1]), _full(s), _full(q)],
        out_specs=_sq3(co, N),
        out_shape=jax.ShapeDtypeStruct((B, co, N), x.dtype),
        compiler_params=_cparams(),
        cost_estimate=pl.CostEstimate(
            flops=2 * B * N * ci * co + 11 * B * N * co,
            transcendentals=B * N * co,
            bytes_accessed=2 * B * N * ci + 4 * B * N * co),
    )(prev, weights[-1], gammas[-1], betas[-1], s, q)
    return out


# speedup vs baseline: 9.6804x; 1.0067x over previous
import jax
import jax.numpy as jnp
from jax.experimental import pallas as pl
from jax.experimental.pallas import tpu as pltpu

_BN_EPS = 1e-5

_INV_PI = 0.3183098861837907
_PI_F32 = 3.14159265
_S3 = -0.16666667
_S5 = 8.3321608e-3
_S7 = -1.9515296e-4


def _fast_sin(v):
    k = jnp.round(v * _INV_PI)
    r = v - k * _PI_F32
    r2 = r * r
    p = _S7
    p = p * r2 + _S5
    p = p * r2 + _S3
    s = r + r * (r2 * p)
    sbits = jax.lax.shift_left(k.astype(jnp.int32), 31)
    return jax.lax.bitcast_convert_type(
        jax.lax.bitcast_convert_type(s, jnp.int32) ^ sbits, jnp.float32)


_BB = 8


def _acc_stats(sums, sq, s_ref, q_ref):
    b = pl.program_id(0)

    @pl.when(b == 0)
    def _init():
        s_ref[...] = sums
        q_ref[...] = sq

    @pl.when(b > 0)
    def _acc():
        s_ref[...] += sums
        q_ref[...] += sq


def _bn(sp_ref, qp_ref, g_ref, b_ref, inv_m):
    mean = sp_ref[...] * inv_m
    ey2 = qp_ref[...] * inv_m
    var = jnp.maximum(ey2 - mean * mean, 0.0)
    scale = g_ref[...] * jax.lax.rsqrt(var + _BN_EPS)
    shift = b_ref[...] - mean * scale
    return scale, shift


def _k_first(x_ref, w_ref, s_ref, q_ref):
    w = w_ref[...]
    sums, sq = 0.0, 0.0
    for j in range(_BB):
        y = jnp.dot(w, x_ref[j], preferred_element_type=jnp.float32)
        sums += jnp.sum(y, axis=1, keepdims=True)
        sq += jnp.sum(y * y, axis=1, keepdims=True)
    _acc_stats(sums, sq, s_ref, q_ref)


def _make_mid(inv_m):
    def _k_mid(prev_ref, wf_ref, g_ref, b_ref, sp_ref, qp_ref, wn_ref,
               act_ref, s_ref, q_ref):
        scale, shift = _bn(sp_ref, qp_ref, g_ref, b_ref, inv_m)
        w_eff = wf_ref[...] * scale
        wn = wn_ref[...]
        sums, sq = 0.0, 0.0
        for j in range(_BB):
            a = _fast_sin(
                jnp.dot(w_eff, prev_ref[j].astype(jnp.float32),
                        preferred_element_type=jnp.float32) + shift)
            act_ref[j] = a.astype(act_ref.dtype)
            y = jnp.dot(wn, a, preferred_element_type=jnp.float32)
            sums += jnp.sum(y, axis=1, keepdims=True)
            sq += jnp.sum(y * y, axis=1, keepdims=True)
        _acc_stats(sums, sq, s_ref, q_ref)
    return _k_mid


def _make_last(inv_m):
    def _k_last(prev_ref, wf_ref, g_ref, b_ref, sp_ref, qp_ref, o_ref):
        scale, shift = _bn(sp_ref, qp_ref, g_ref, b_ref, inv_m)
        w_eff = wf_ref[...] * scale
        for j in range(_BB):
            o_ref[j] = _fast_sin(
                jnp.dot(w_eff, prev_ref[j].astype(jnp.float32),
                        preferred_element_type=jnp.float32) + shift
            ).astype(o_ref.dtype)
    return _k_last


def _cparams():
    return pltpu.CompilerParams(dimension_semantics=("arbitrary",),
                                vmem_limit_bytes=58 * 1024 * 1024)


def kernel(x, W0, b0, gamma0, beta0, W1, b1, gamma1, beta1,
           W2, b2, gamma2, beta2, W3, b3, gamma3, beta3):
    B, C0, N = (int(s) for s in x.shape)
    weights = [W0, W1, W2, W3]
    gammas = [gamma0, gamma1, gamma2, gamma3]
    betas = [beta0, beta1, beta2, beta3]
    couts = [int(w.shape[0]) for w in weights]
    cins = [int(w.shape[1]) for w in weights]
    assert cins[0] == C0 and N % 128 == 0 and C0 % 8 == 0 and B % _BB == 0
    inv_m = 1.0 / float(B * N)
    n_layers = len(weights)
    grid = (B // _BB,)

    def _sq3(c, n):
        return pl.BlockSpec((_BB, c, n), lambda b: (b, 0, 0))

    def _full(arr):
        shape = tuple(int(s) for s in arr.shape)
        return pl.BlockSpec(shape, lambda b: (0,) * len(shape))

    def stat_outs(c):
        shapes = [jax.ShapeDtypeStruct((c, 1), jnp.float32)] * 2
        specs = [_full_shape((c, 1))] * 2
        return shapes, specs

    def _full_shape(shape):
        return pl.BlockSpec(shape, lambda b: (0,) * len(shape))

    st_sh, st_sp = stat_outs(couts[0])
    s, q = pl.pallas_call(
        _k_first,
        grid=grid,
        in_specs=[_sq3(C0, N), _full(W0)],
        out_shape=st_sh,
        out_specs=st_sp,
        compiler_params=_cparams(),
        cost_estimate=pl.CostEstimate(
            flops=2 * B * N * C0 * couts[0] + 3 * B * N * couts[0],
            transcendentals=0,
            bytes_accessed=4 * B * C0 * N),
    )(x, W0)

    prev = x
    for i in range(n_layers - 1):
        ci, co, cn = cins[i], couts[i], couts[i + 1]
        st_sh, st_sp = stat_outs(cn)
        prev, s, q = pl.pallas_call(
            _make_mid(inv_m),
            grid=grid,
            in_specs=[_sq3(ci, N), _full(weights[i]), _full(gammas[i]),
                      _full(betas[i]), _full(s), _full(q),
                      _full(weights[i + 1])],
            out_shape=[jax.ShapeDtypeStruct((B, co, N), jnp.bfloat16)] + st_sh,
            out_specs=[_sq3(co, N)] + st_sp,
            compiler_params=_cparams(),
            cost_estimate=pl.CostEstimate(
                flops=2 * B * N * (ci * co + co * cn) + 11 * B * N * co,
                transcendentals=B * N * co,
                bytes_accessed=4 * B * N * ci // (1 if i == 0 else 2)
                + 2 * B * N * co),
        )(prev, weights[i], gammas[i], betas[i], s, q, weights[i + 1])

    ci, co = cins[-1], couts[-1]
    out = pl.pallas_call(
        _make_last(inv_m),
        grid=grid,
        in_specs=[_sq3(ci, N), _full(weights[-1]), _full(gammas[-1]),
                  _full(betas[-1]), _full(s), _full(q)],
        out_specs=_sq3(co, N),
        out_shape=jax.ShapeDtypeStruct((B, co, N), x.dtype),
        compiler_params=_cparams(),
        cost_estimate=pl.CostEstimate(
            flops=2 * B * N * ci * co + 11 * B * N * co,
            transcendentals=B * N * co,
            bytes_accessed=2 * B * N * ci + 4 * B * N * co),
    )(prev, weights[-1], gammas[-1], betas[-1], s, q)
    return out


# speedup vs baseline: 9.8698x; 1.0196x over previous
import jax
import jax.numpy as jnp
from jax.experimental import pallas as pl
from jax.experimental.pallas import tpu as pltpu

_BN_EPS = 1e-5

_INV_PI = 0.3183098861837907
_PI_F32 = 3.14159265
_S3 = -0.16666667
_S5 = 8.3321608e-3
_S7 = -1.9515296e-4

_BB = 8


def _fast_sin(v):
    k = jnp.round(v * _INV_PI)
    r = v - k * _PI_F32
    r2 = r * r
    p = _S7
    p = p * r2 + _S5
    p = p * r2 + _S3
    s = r + r * (r2 * p)
    sbits = jax.lax.shift_left(k.astype(jnp.int32), 31)
    return jax.lax.bitcast_convert_type(
        jax.lax.bitcast_convert_type(s, jnp.int32) ^ sbits, jnp.float32)


def _gram(a):
    return jax.lax.dot_general(a, a, (((1,), (1,)), ((), ())),
                               preferred_element_type=jnp.float32)


def _acc_stats(sums, gram, s_ref, g_ref):
    b = pl.program_id(0)

    @pl.when(b == 0)
    def _init():
        s_ref[...] = sums
        g_ref[...] = gram

    @pl.when(b > 0)
    def _acc():
        s_ref[...] += sums
        g_ref[...] += gram


def _bn(sp_ref, gp_ref, g_ref, b_ref, w_ref, inv_m):
    w = w_ref[...]
    mean = jnp.dot(w, sp_ref[...], preferred_element_type=jnp.float32) * inv_m
    t = jnp.dot(w, gp_ref[...], preferred_element_type=jnp.float32)
    ey2 = jnp.sum(t * w, axis=1, keepdims=True) * inv_m
    var = jnp.maximum(ey2 - mean * mean, 0.0)
    scale = g_ref[...] * jax.lax.rsqrt(var + _BN_EPS)
    shift = b_ref[...] - mean * scale
    return scale, shift


def _k_first(x_ref, s_ref, g_ref):
    sums, gram = 0.0, 0.0
    for j in range(_BB):
        xj = x_ref[j]
        sums += jnp.sum(xj, axis=1, keepdims=True)
        gram += _gram(xj)
    _acc_stats(sums, gram, s_ref, g_ref)


def _make_mid(inv_m):
    def _k_mid(prev_ref, wf_ref, g_ref, b_ref, sp_ref, gp_ref,
               act_ref, s_ref, g_out_ref):
        scale, shift = _bn(sp_ref, gp_ref, g_ref, b_ref, wf_ref, inv_m)
        w_eff = wf_ref[...] * scale
        sums, gram = 0.0, 0.0
        for j in range(_BB):
            a = _fast_sin(
                jnp.dot(w_eff, prev_ref[j].astype(jnp.float32),
                        preferred_element_type=jnp.float32) + shift)
            act_ref[j] = a.astype(act_ref.dtype)
            sums += jnp.sum(a, axis=1, keepdims=True)
            gram += _gram(a)
        _acc_stats(sums, gram, s_ref, g_out_ref)
    return _k_mid


def _make_last(inv_m):
    def _k_last(prev_ref, wf_ref, g_ref, b_ref, sp_ref, gp_ref, o_ref):
        scale, shift = _bn(sp_ref, gp_ref, g_ref, b_ref, wf_ref, inv_m)
        w_eff = wf_ref[...] * scale
        for j in range(_BB):
            o_ref[j] = _fast_sin(
                jnp.dot(w_eff, prev_ref[j].astype(jnp.float32),
                        preferred_element_type=jnp.float32) + shift
            ).astype(o_ref.dtype)
    return _k_last


def _cparams():
    return pltpu.CompilerParams(dimension_semantics=("arbitrary",),
                                vmem_limit_bytes=58 * 1024 * 1024)


def kernel(x, W0, b0, gamma0, beta0, W1, b1, gamma1, beta1,
           W2, b2, gamma2, beta2, W3, b3, gamma3, beta3):
    B, C0, N = (int(s) for s in x.shape)
    weights = [W0, W1, W2, W3]
    gammas = [gamma0, gamma1, gamma2, gamma3]
    betas = [beta0, beta1, beta2, beta3]
    couts = [int(w.shape[0]) for w in weights]
    cins = [int(w.shape[1]) for w in weights]
    assert cins[0] == C0 and N % 128 == 0 and C0 % 8 == 0 and B % _BB == 0
    inv_m = 1.0 / float(B * N)
    n_layers = len(weights)
    grid = (B // _BB,)

    def _blk3(c, n):
        return pl.BlockSpec((_BB, c, n), lambda b: (b, 0, 0))

    def _full(arr):
        shape = tuple(int(s) for s in arr.shape)
        return pl.BlockSpec(shape, lambda b: (0,) * len(shape))

    def _full_shape(shape):
        return pl.BlockSpec(shape, lambda b: (0,) * len(shape))

    def stat_outs(c):
        shapes = [jax.ShapeDtypeStruct((c, 1), jnp.float32),
                  jax.ShapeDtypeStruct((c, c), jnp.float32)]
        specs = [_full_shape((c, 1)), _full_shape((c, c))]
        return shapes, specs

    st_sh, st_sp = stat_outs(C0)
    s, g = pl.pallas_call(
        _k_first,
        grid=grid,
        in_specs=[_blk3(C0, N)],
        out_shape=st_sh,
        out_specs=st_sp,
        compiler_params=_cparams(),
        cost_estimate=pl.CostEstimate(
            flops=2 * B * N * C0 * C0 + B * N * C0,
            transcendentals=0,
            bytes_accessed=4 * B * C0 * N),
    )(x)

    prev = x
    for i in range(n_layers - 1):
        ci, co = cins[i], couts[i]
        st_sh, st_sp = stat_outs(co)
        prev, s, g = pl.pallas_call(
            _make_mid(inv_m),
            grid=grid,
            in_specs=[_blk3(ci, N), _full(weights[i]), _full(gammas[i]),
                      _full(betas[i]), _full_shape((ci, 1)),
                      _full_shape((ci, ci))],
            out_shape=[jax.ShapeDtypeStruct((B, co, N), jnp.bfloat16)] + st_sh,
            out_specs=[_blk3(co, N)] + st_sp,
            compiler_params=_cparams(),
            cost_estimate=pl.CostEstimate(
                flops=2 * B * N * (ci * co + co * co) + 11 * B * N * co,
                transcendentals=B * N * co,
                bytes_accessed=4 * B * N * ci // (1 if i == 0 else 2)
                + 2 * B * N * co),
        )(prev, weights[i], gammas[i], betas[i], s, g)

    ci, co = cins[-1], couts[-1]
    out = pl.pallas_call(
        _make_last(inv_m),
        grid=grid,
        in_specs=[_blk3(ci, N), _full(weights[-1]), _full(gammas[-1]),
                  _full(betas[-1]), _full_shape((ci, 1)),
                  _full_shape((ci, ci))],
        out_specs=_blk3(co, N),
        out_shape=jax.ShapeDtypeStruct((B, co, N), x.dtype),
        compiler_params=_cparams(),
        cost_estimate=pl.CostEstimate(
            flops=2 * B * N * ci * co + 11 * B * N * co,
            transcendentals=B * N * co,
            bytes_accessed=2 * B * N * ci + 4 * B * N * co),
    )(prev, weights[-1], gammas[-1], betas[-1], s, g)
    return out
